# baseline (speedup 1.0000x reference)
import functools

import jax
import jax.numpy as jnp
from jax import lax
from jax.experimental import pallas as pl
from jax.experimental.pallas import tpu as pltpu
from jax.experimental.pallas import tpu_sc as plsc

N, D, S, B, H, O = 100000, 128, 16, 2048, 128, 64
NC, NS = 2, 16
NW = NC * NS
BPW = B // NW
LANES = 16
NIDS = BPW * S
FCH = 4
CSEED = BPW // FCH
FROWS = CSEED * S


def _sc_gather_mean(nodes, node_features, nidx_sm):
    mesh = plsc.VectorSubcoreMesh(core_axis_name="c", subcore_axis_name="s")

    @functools.partial(
        pl.kernel,
        out_type=(
            jax.ShapeDtypeStruct((B, D), jnp.float32),
            jax.ShapeDtypeStruct((B, D), jnp.float32),
        ),
        mesh=mesh,
        compiler_params=pltpu.CompilerParams(use_tc_tiling_on_sc=False),
        scratch_types=[
            pltpu.VMEM((BPW,), jnp.int32),
            pltpu.VMEM((NIDS,), jnp.int32),
            pltpu.VMEM((NIDS,), jnp.int32),
            pltpu.VMEM((BPW, D), jnp.float32),
            pltpu.VMEM((FROWS, D), jnp.float32),
            pltpu.VMEM((FROWS, D), jnp.float32),
            pltpu.VMEM((BPW, D), jnp.float32),
            pltpu.SemaphoreType.DMA,
            pltpu.SemaphoreType.DMA,
            pltpu.SemaphoreType.DMA,
            pltpu.SemaphoreType.DMA,
            pltpu.SemaphoreType.DMA,
            pltpu.SemaphoreType.DMA,
            pltpu.SemaphoreType.DMA,
            pltpu.SemaphoreType.DMA,
        ],
    )
    def k(nodes_hbm, feats_hbm, nidx_hbm, orig_out, agg_out,
          seeds_v, cidx_v, flat_v, orig_v, gbuf0, gbuf1, agg_v,
          semi0, semi1, semi2, semi3, semf0, semf1, semo, semw):
        wid = lax.axis_index("s") * NC + lax.axis_index("c")
        base = wid * BPW

        pltpu.sync_copy(nodes_hbm.at[pl.ds(base, BPW)], seeds_v)
        origather = pltpu.async_copy(feats_hbm.at[seeds_v], orig_v, semo)

        jvec = lax.iota(jnp.int32, LANES) * N
        semis = (semi0, semi1, semi2, semi3)

        def idpiece(c, fire=True):
            mk = pltpu.async_copy if fire else pltpu.make_async_copy
            return mk(
                nidx_hbm.at[cidx_v.at[pl.ds(c * FROWS, FROWS)]],
                flat_v.at[pl.ds(c * FROWS, FROWS)], semis[c],
            )

        for g in range(FCH):
            svec = seeds_v[pl.ds(g * CSEED, CSEED)]
            for i in range(CSEED):
                cidx_v[pl.ds((g * CSEED + i) * S, S)] = svec[i] + jvec
            idpiece(g)

        bufs = (gbuf0, gbuf1)
        semfs = (semf0, semf1)

        def fchunk(c, fire=True):
            mk = pltpu.async_copy if fire else pltpu.make_async_copy
            return mk(
                feats_hbm.at[flat_v.at[pl.ds(c * FROWS, FROWS)]],
                bufs[c % 2], semfs[c % 2],
            )

        inv = jnp.float32(1.0 / S)
        writes = []
        idpiece(0, fire=False).wait()
        fchunk(0)
        idpiece(1, fire=False).wait()
        fchunk(1)
        for c in range(FCH):
            buf = bufs[c % 2]
            fchunk(c, fire=False).wait()

            def acc_body(si, carry, buf=buf, c=c):
                r0 = si * S
                for kk in range(D // LANES):
                    col = pl.ds(kk * LANES, LANES)
                    v = buf[r0, col]
                    for j in range(1, S):
                        v = v + buf[r0 + j, col]
                    agg_v[c * CSEED + si, col] = v * inv
                return carry

            lax.fori_loop(0, CSEED, acc_body, 0)
            writes.append(pltpu.async_copy(
                agg_v.at[pl.ds(c * CSEED, CSEED)],
                agg_out.at[pl.ds(base + c * CSEED, CSEED)], semw,
            ))
            if c + 2 < FCH:
                idpiece(c + 2, fire=False).wait()
                fchunk(c + 2)

        origather.wait()
        pltpu.sync_copy(orig_v, orig_out.at[pl.ds(base, BPW)])
        for w in writes:
            w.wait()

    return k(nodes, node_features, nidx_sm)


def _tc_dense(orig, agg, W2, Wout):
    BM = 1024
    dn = (((1,), (1,)), ((), ()))

    def body(o_ref, a_ref, w2_ref, wout_ref, out_ref):
        h = lax.dot_general(o_ref[...], w2_ref[:, :D], dn,
                            preferred_element_type=jnp.float32)
        h = h + lax.dot_general(a_ref[...], w2_ref[:, D:], dn,
                                preferred_element_type=jnp.float32)
        h = jnp.maximum(h, 0.0)
        logits = lax.dot_general(wout_ref[...], h, dn,
                                 preferred_element_type=jnp.float32)
        m = jnp.max(logits, axis=0, keepdims=True)
        e = jnp.exp(logits - m)
        out_ref[...] = e / jnp.sum(e, axis=0, keepdims=True)

    out_t = pl.pallas_call(
        body,
        grid=(B // BM,),
        in_specs=[
            pl.BlockSpec((BM, D), lambda i: (i, 0)),
            pl.BlockSpec((BM, D), lambda i: (i, 0)),
            pl.BlockSpec((H, 2 * D), lambda i: (0, 0)),
            pl.BlockSpec((O, H), lambda i: (0, 0)),
        ],
        out_specs=pl.BlockSpec((O, BM), lambda i: (0, i)),
        out_shape=jax.ShapeDtypeStruct((O, B), jnp.float32),
    )(orig, agg, W2, Wout)
    return jnp.transpose(out_t)


def kernel(nodes, node_features, neigh_idx, W1, W2, Wout):
    nodes = nodes.astype(jnp.int32)
    nidx_sm = jnp.transpose(neigh_idx.astype(jnp.int32)).reshape(N * S)
    orig, agg = _sc_gather_mean(nodes, node_features, nidx_sm)
    return _tc_dense(orig, agg, W2, Wout)


# speedup vs baseline: 1.1386x; 1.1386x over previous
import functools

import jax
import jax.numpy as jnp
from jax import lax
from jax.experimental import pallas as pl
from jax.experimental.pallas import tpu as pltpu
from jax.experimental.pallas import tpu_sc as plsc

N, D, S, B, H, O = 100000, 128, 16, 2048, 128, 64
NC, NS = 2, 16
NW = NC * NS
BPW = B // NW
LANES = 16
NIDS = BPW * S
FCH = 4
CSEED = BPW // FCH
FROWS = CSEED * S


def _sc_gather_mean(nodes, node_features, nidx_sm):
    mesh = plsc.VectorSubcoreMesh(core_axis_name="c", subcore_axis_name="s")

    @functools.partial(
        pl.kernel,
        out_type=(
            jax.ShapeDtypeStruct((B, D), jnp.float32),
            jax.ShapeDtypeStruct((B, D), jnp.float32),
        ),
        mesh=mesh,
        compiler_params=pltpu.CompilerParams(use_tc_tiling_on_sc=False),
        scratch_types=[
            pltpu.VMEM((BPW,), jnp.int32),
            pltpu.VMEM((NIDS,), jnp.int32),
            pltpu.VMEM((NIDS,), jnp.int32),
            pltpu.VMEM((BPW, D), jnp.float32),
            pltpu.VMEM((FROWS, D), jnp.float32),
            pltpu.VMEM((FROWS, D), jnp.float32),
            pltpu.VMEM((BPW, D), jnp.float32),
            pltpu.SemaphoreType.DMA,
            pltpu.SemaphoreType.DMA,
            pltpu.SemaphoreType.DMA,
            pltpu.SemaphoreType.DMA,
        ],
    )
    def k(nodes_hbm, feats_hbm, nidx_hbm, orig_out, agg_out,
          seeds_v, cidx_v, flat_v, orig_v, gbuf0, gbuf1, acc_v,
          semi, semo, semf0, semf1):
        wid = lax.axis_index("s") * NC + lax.axis_index("c")
        base = wid * BPW

        pltpu.sync_copy(nodes_hbm.at[pl.ds(base, BPW)], seeds_v)

        for g in range(BPW // LANES):
            nvec = seeds_v[pl.ds(g * LANES, LANES)]
            for j in range(S):
                cidx_v[pl.ds(j * BPW + g * LANES, LANES)] = nvec + (j * N)

        idgather = pltpu.async_copy(nidx_hbm.at[cidx_v], flat_v, semi)
        origather = pltpu.async_copy(feats_hbm.at[seeds_v], orig_v, semo)
        idgather.wait()

        bufs = (gbuf0, gbuf1)
        sems = (semf0, semf1)

        def fire(c):
            pltpu.async_copy(
                feats_hbm.at[flat_v.at[pl.ds(c * FROWS, FROWS)]],
                bufs[c % 2], sems[c % 2],
            )

        fire(0)
        fire(1)
        inv = jnp.float32(1.0 / S)
        for c in range(FCH):
            buf = bufs[c % 2]
            pltpu.make_async_copy(
                feats_hbm.at[flat_v.at[pl.ds(c * FROWS, FROWS)]],
                buf, sems[c % 2],
            ).wait()

            if c == 0:
                def acc0_body(si, carry):
                    for kk in range(D // LANES):
                        col = pl.ds(kk * LANES, LANES)
                        v = (buf[si, col] + buf[BPW + si, col]
                             + buf[2 * BPW + si, col] + buf[3 * BPW + si, col])
                        acc_v[si, col] = v
                    return carry
                lax.fori_loop(0, BPW, acc0_body, 0)
            else:
                def accn_body(si, carry):
                    for kk in range(D // LANES):
                        col = pl.ds(kk * LANES, LANES)
                        v = (buf[si, col] + buf[BPW + si, col]
                             + buf[2 * BPW + si, col] + buf[3 * BPW + si, col])
                        acc_v[si, col] = acc_v[si, col] + v
                    return carry
                lax.fori_loop(0, BPW, accn_body, 0)

            if c + 2 < FCH:
                fire(c + 2)

        origather.wait()
        pltpu.sync_copy(orig_v, orig_out.at[pl.ds(base, BPW)])

        def scale_body(si, carry):
            for kk in range(D // LANES):
                col = pl.ds(kk * LANES, LANES)
                acc_v[si, col] = acc_v[si, col] * inv
            return carry

        lax.fori_loop(0, BPW, scale_body, 0)
        pltpu.sync_copy(acc_v, agg_out.at[pl.ds(base, BPW)])

    return k(nodes, node_features, nidx_sm)


def _tc_dense(orig, agg, W2, Wout):
    BM = 1024
    dn = (((1,), (1,)), ((), ()))

    def body(o_ref, a_ref, w2_ref, wout_ref, out_ref):
        h = lax.dot_general(o_ref[...], w2_ref[:, :D], dn,
                            preferred_element_type=jnp.float32)
        h = h + lax.dot_general(a_ref[...], w2_ref[:, D:], dn,
                                preferred_element_type=jnp.float32)
        h = jnp.maximum(h, 0.0)
        logits = lax.dot_general(wout_ref[...], h, dn,
                                 preferred_element_type=jnp.float32)
        m = jnp.max(logits, axis=0, keepdims=True)
        e = jnp.exp(logits - m)
        out_ref[...] = e / jnp.sum(e, axis=0, keepdims=True)

    out_t = pl.pallas_call(
        body,
        grid=(B // BM,),
        in_specs=[
            pl.BlockSpec((BM, D), lambda i: (i, 0)),
            pl.BlockSpec((BM, D), lambda i: (i, 0)),
            pl.BlockSpec((H, 2 * D), lambda i: (0, 0)),
            pl.BlockSpec((O, H), lambda i: (0, 0)),
        ],
        out_specs=pl.BlockSpec((O, BM), lambda i: (0, i)),
        out_shape=jax.ShapeDtypeStruct((O, B), jnp.float32),
    )(orig, agg, W2, Wout)
    return jnp.transpose(out_t)


def kernel(nodes, node_features, neigh_idx, W1, W2, Wout):
    nodes = nodes.astype(jnp.int32)
    nidx_sm = jnp.transpose(neigh_idx.astype(jnp.int32)).reshape(N * S)
    orig, agg = _sc_gather_mean(nodes, node_features, nidx_sm)
    return _tc_dense(orig, agg, W2, Wout)


# speedup vs baseline: 1.1642x; 1.0225x over previous
import functools

import jax
import jax.numpy as jnp
from jax import lax
from jax.experimental import pallas as pl
from jax.experimental.pallas import tpu as pltpu
from jax.experimental.pallas import tpu_sc as plsc

N, D, S, B, H, O = 100000, 128, 16, 2048, 128, 64
NC, NS = 2, 16
NW = NC * NS
BPW = B // NW
LANES = 16
NIDS = BPW * S
FCH = 4
CSEED = BPW // FCH
FROWS = CSEED * S


def _sc_gather_mean(nodes, node_features, nidx_sm):
    mesh = plsc.VectorSubcoreMesh(core_axis_name="c", subcore_axis_name="s")

    @functools.partial(
        pl.kernel,
        out_type=(
            jax.ShapeDtypeStruct((B, D), jnp.float32),
            jax.ShapeDtypeStruct((B, D), jnp.float32),
        ),
        mesh=mesh,
        compiler_params=pltpu.CompilerParams(use_tc_tiling_on_sc=False),
        scratch_types=[
            pltpu.VMEM((BPW,), jnp.int32),
            pltpu.VMEM((NIDS,), jnp.int32),
            pltpu.VMEM((NIDS,), jnp.int32),
            pltpu.VMEM((BPW, D), jnp.float32),
            pltpu.VMEM((FROWS, D), jnp.float32),
            pltpu.VMEM((FROWS, D), jnp.float32),
            pltpu.VMEM((FROWS, D), jnp.float32),
            pltpu.VMEM((BPW, D), jnp.float32),
            pltpu.SemaphoreType.DMA,
            pltpu.SemaphoreType.DMA,
            pltpu.SemaphoreType.DMA,
            pltpu.SemaphoreType.DMA,
            pltpu.SemaphoreType.DMA,
            pltpu.SemaphoreType.DMA,
        ],
    )
    def k(nodes_hbm, feats_hbm, nidx_hbm, orig_out, agg_out,
          seeds_v, cidx_v, flat_v, orig_v, gbuf0, gbuf1, gbuf2, acc_v,
          semia, semib, semo, semf0, semf1, semf2):
        wid = lax.axis_index("s") * NC + lax.axis_index("c")
        base = wid * BPW

        pltpu.sync_copy(nodes_hbm.at[pl.ds(base, BPW)], seeds_v)

        for g in range(BPW // LANES):
            nvec = seeds_v[pl.ds(g * LANES, LANES)]
            for j in range(S):
                cidx_v[pl.ds(j * BPW + g * LANES, LANES)] = nvec + (j * N)

        def idpiece(lo, ln, sem, fire):
            mk = pltpu.async_copy if fire else pltpu.make_async_copy
            return mk(nidx_hbm.at[cidx_v.at[pl.ds(lo, ln)]],
                      flat_v.at[pl.ds(lo, ln)], sem)

        idpiece(0, FROWS, semia, True)
        idpiece(FROWS, NIDS - FROWS, semib, True)
        origather = pltpu.async_copy(feats_hbm.at[seeds_v], orig_v, semo)

        bufs = (gbuf0, gbuf1, gbuf2)
        sems = (semf0, semf1, semf2)

        def fchunk(c, fire=True):
            mk = pltpu.async_copy if fire else pltpu.make_async_copy
            return mk(
                feats_hbm.at[flat_v.at[pl.ds(c * FROWS, FROWS)]],
                bufs[c % 3], sems[c % 3],
            )

        idpiece(0, FROWS, semia, False).wait()
        fchunk(0)
        idpiece(FROWS, NIDS - FROWS, semib, False).wait()
        fchunk(1)
        fchunk(2)
        inv = jnp.float32(1.0 / S)
        for c in range(FCH):
            buf = bufs[c % 3]
            fchunk(c, fire=False).wait()

            if c == 0:
                def acc0_body(si, carry, buf=buf):
                    for kk in range(D // LANES):
                        col = pl.ds(kk * LANES, LANES)
                        v = (buf[si, col] + buf[BPW + si, col]
                             + buf[2 * BPW + si, col] + buf[3 * BPW + si, col])
                        acc_v[si, col] = v
                    return carry
                lax.fori_loop(0, BPW, acc0_body, 0)
            elif c < FCH - 1:
                def accn_body(si, carry, buf=buf):
                    for kk in range(D // LANES):
                        col = pl.ds(kk * LANES, LANES)
                        v = (buf[si, col] + buf[BPW + si, col]
                             + buf[2 * BPW + si, col] + buf[3 * BPW + si, col])
                        acc_v[si, col] = acc_v[si, col] + v
                    return carry
                lax.fori_loop(0, BPW, accn_body, 0)
            else:
                def accl_body(si, carry, buf=buf):
                    for kk in range(D // LANES):
                        col = pl.ds(kk * LANES, LANES)
                        v = (buf[si, col] + buf[BPW + si, col]
                             + buf[2 * BPW + si, col] + buf[3 * BPW + si, col])
                        acc_v[si, col] = (acc_v[si, col] + v) * inv
                    return carry
                lax.fori_loop(0, BPW, accl_body, 0)

            if c + 3 < FCH:
                fchunk(c + 3)

        origather.wait()
        pltpu.sync_copy(orig_v, orig_out.at[pl.ds(base, BPW)])
        pltpu.sync_copy(acc_v, agg_out.at[pl.ds(base, BPW)])

    return k(nodes, node_features, nidx_sm)


def _tc_dense(orig, agg, W2, Wout):
    BM = 1024
    dn = (((1,), (1,)), ((), ()))

    def body(o_ref, a_ref, w2_ref, wout_ref, out_ref):
        h = lax.dot_general(o_ref[...], w2_ref[:, :D], dn,
                            preferred_element_type=jnp.float32)
        h = h + lax.dot_general(a_ref[...], w2_ref[:, D:], dn,
                                preferred_element_type=jnp.float32)
        h = jnp.maximum(h, 0.0)
        logits = lax.dot_general(wout_ref[...], h, dn,
                                 preferred_element_type=jnp.float32)
        m = jnp.max(logits, axis=0, keepdims=True)
        e = jnp.exp(logits - m)
        out_ref[...] = e / jnp.sum(e, axis=0, keepdims=True)

    out_t = pl.pallas_call(
        body,
        grid=(B // BM,),
        in_specs=[
            pl.BlockSpec((BM, D), lambda i: (i, 0)),
            pl.BlockSpec((BM, D), lambda i: (i, 0)),
            pl.BlockSpec((H, 2 * D), lambda i: (0, 0)),
            pl.BlockSpec((O, H), lambda i: (0, 0)),
        ],
        out_specs=pl.BlockSpec((O, BM), lambda i: (0, i)),
        out_shape=jax.ShapeDtypeStruct((O, B), jnp.float32),
    )(orig, agg, W2, Wout)
    return jnp.transpose(out_t)


def kernel(nodes, node_features, neigh_idx, W1, W2, Wout):
    nodes = nodes.astype(jnp.int32)
    nidx_sm = jnp.transpose(neigh_idx.astype(jnp.int32)).reshape(N * S)
    orig, agg = _sc_gather_mean(nodes, node_features, nidx_sm)
    return _tc_dense(orig, agg, W2, Wout)


# speedup vs baseline: 1.1659x; 1.0014x over previous
import functools

import jax
import jax.numpy as jnp
from jax import lax
from jax.experimental import pallas as pl
from jax.experimental.pallas import tpu as pltpu
from jax.experimental.pallas import tpu_sc as plsc

N, D, S, B, H, O = 100000, 128, 16, 2048, 128, 64
NC, NS = 2, 16
NW = NC * NS
BPW = B // NW
LANES = 16
NIDS = BPW * S
FCH = 4
CSEED = BPW // FCH
FROWS = CSEED * S


def _sc_gather_mean(nodes, node_features, nidx_sm):
    mesh = plsc.VectorSubcoreMesh(core_axis_name="c", subcore_axis_name="s")

    @functools.partial(
        pl.kernel,
        out_type=(
            jax.ShapeDtypeStruct((B, D), jnp.float32),
            jax.ShapeDtypeStruct((B, D), jnp.float32),
        ),
        mesh=mesh,
        compiler_params=pltpu.CompilerParams(use_tc_tiling_on_sc=False),
        scratch_types=[
            pltpu.VMEM((BPW,), jnp.int32),
            pltpu.VMEM((NIDS,), jnp.int32),
            pltpu.VMEM((NIDS,), jnp.int32),
            pltpu.VMEM((BPW, D), jnp.float32),
            pltpu.VMEM((FROWS, D), jnp.float32),
            pltpu.VMEM((FROWS, D), jnp.float32),
            pltpu.VMEM((FROWS, D), jnp.float32),
            pltpu.VMEM((BPW, D), jnp.float32),
            pltpu.SemaphoreType.DMA,
            pltpu.SemaphoreType.DMA,
            pltpu.SemaphoreType.DMA,
            pltpu.SemaphoreType.DMA,
            pltpu.SemaphoreType.DMA,
            pltpu.SemaphoreType.DMA,
        ],
    )
    def k(nodes_hbm, feats_hbm, nidx_hbm, orig_out, agg_out,
          seeds_v, cidx_v, flat_v, orig_v, gbuf0, gbuf1, gbuf2, acc_v,
          semia, semib, semo, semf0, semf1, semf2):
        wid = lax.axis_index("s") * NC + lax.axis_index("c")
        base = wid * BPW

        pltpu.sync_copy(nodes_hbm.at[pl.ds(base, BPW)], seeds_v)

        for g in range(BPW // LANES):
            nvec = seeds_v[pl.ds(g * LANES, LANES)]
            for j in range(S):
                cidx_v[pl.ds(j * BPW + g * LANES, LANES)] = nvec + (j * N)

        def idpiece(lo, ln, sem, fire):
            mk = pltpu.async_copy if fire else pltpu.make_async_copy
            return mk(nidx_hbm.at[cidx_v.at[pl.ds(lo, ln)]],
                      flat_v.at[pl.ds(lo, ln)], sem)

        idpiece(0, FROWS, semia, True)
        idpiece(FROWS, NIDS - FROWS, semib, True)
        origather = pltpu.async_copy(feats_hbm.at[seeds_v], orig_v, semo)

        bufs = (gbuf0, gbuf1, gbuf2)
        sems = (semf0, semf1, semf2)

        def fchunk(c, fire=True):
            mk = pltpu.async_copy if fire else pltpu.make_async_copy
            return mk(
                feats_hbm.at[flat_v.at[pl.ds(c * FROWS, FROWS)]],
                bufs[c % 3], sems[c % 3],
            )

        idpiece(0, FROWS, semia, False).wait()
        fchunk(0)
        idpiece(FROWS, NIDS - FROWS, semib, False).wait()
        fchunk(1)
        fchunk(2)
        origather.wait()
        pltpu.sync_copy(orig_v, orig_out.at[pl.ds(base, BPW)])
        inv = jnp.float32(1.0 / S)
        for c in range(FCH):
            buf = bufs[c % 3]
            fchunk(c, fire=False).wait()

            if c == 0:
                def acc0_body(si, carry, buf=buf):
                    for kk in range(D // LANES):
                        col = pl.ds(kk * LANES, LANES)
                        v = (buf[si, col] + buf[BPW + si, col]
                             + buf[2 * BPW + si, col] + buf[3 * BPW + si, col])
                        acc_v[si, col] = v
                    return carry
                lax.fori_loop(0, BPW, acc0_body, 0)
            elif c < FCH - 1:
                def accn_body(si, carry, buf=buf):
                    for kk in range(D // LANES):
                        col = pl.ds(kk * LANES, LANES)
                        v = (buf[si, col] + buf[BPW + si, col]
                             + buf[2 * BPW + si, col] + buf[3 * BPW + si, col])
                        acc_v[si, col] = acc_v[si, col] + v
                    return carry
                lax.fori_loop(0, BPW, accn_body, 0)
            else:
                def accl_body(si, carry, buf=buf):
                    for kk in range(D // LANES):
                        col = pl.ds(kk * LANES, LANES)
                        v = (buf[si, col] + buf[BPW + si, col]
                             + buf[2 * BPW + si, col] + buf[3 * BPW + si, col])
                        acc_v[si, col] = (acc_v[si, col] + v) * inv
                    return carry
                lax.fori_loop(0, BPW, accl_body, 0)

            if c + 3 < FCH:
                fchunk(c + 3)

        pltpu.sync_copy(acc_v, agg_out.at[pl.ds(base, BPW)])

    return k(nodes, node_features, nidx_sm)


def _tc_dense(orig, agg, W2, Wout):
    BM = 2048
    dn = (((1,), (1,)), ((), ()))

    def body(o_ref, a_ref, w2_ref, wout_ref, out_ref):
        h = lax.dot_general(o_ref[...], w2_ref[:, :D], dn,
                            preferred_element_type=jnp.float32)
        h = h + lax.dot_general(a_ref[...], w2_ref[:, D:], dn,
                                preferred_element_type=jnp.float32)
        h = jnp.maximum(h, 0.0)
        logits = lax.dot_general(wout_ref[...], h, dn,
                                 preferred_element_type=jnp.float32)
        m = jnp.max(logits, axis=0, keepdims=True)
        e = jnp.exp(logits - m)
        out_ref[...] = e / jnp.sum(e, axis=0, keepdims=True)

    out_t = pl.pallas_call(
        body,
        grid=(B // BM,),
        in_specs=[
            pl.BlockSpec((BM, D), lambda i: (i, 0)),
            pl.BlockSpec((BM, D), lambda i: (i, 0)),
            pl.BlockSpec((H, 2 * D), lambda i: (0, 0)),
            pl.BlockSpec((O, H), lambda i: (0, 0)),
        ],
        out_specs=pl.BlockSpec((O, BM), lambda i: (0, i)),
        out_shape=jax.ShapeDtypeStruct((O, B), jnp.float32),
    )(orig, agg, W2, Wout)
    return jnp.transpose(out_t)


def kernel(nodes, node_features, neigh_idx, W1, W2, Wout):
    nodes = nodes.astype(jnp.int32)
    nidx_sm = jnp.transpose(neigh_idx.astype(jnp.int32)).reshape(N * S)
    orig, agg = _sc_gather_mean(nodes, node_features, nidx_sm)
    return _tc_dense(orig, agg, W2, Wout)
